# Initial kernel scaffold; baseline (speedup 1.0000x reference)
import jax
import jax.numpy as jnp
from jax.experimental import pallas as pl


def kernel(inputs, pos_table):
    raise NotImplementedError("write your pallas kernel here")


# speedup vs baseline: 1.7200x; 1.7200x over previous
import jax
import jax.numpy as jnp
from jax.experimental import pallas as pl


def _add_body(in_ref, pos_ref, out_ref):
    out_ref[...] = in_ref[...] + pos_ref[...][None, :, :]


def kernel(inputs, pos_table):
    B, S, D = inputs.shape
    BS = 256
    grid = (S // BS,)
    return pl.pallas_call(
        _add_body,
        grid=grid,
        in_specs=[
            pl.BlockSpec((B, BS, D), lambda i: (0, i, 0)),
            pl.BlockSpec((BS, D), lambda i: (i, 0)),
        ],
        out_specs=pl.BlockSpec((B, BS, D), lambda i: (0, i, 0)),
        out_shape=jax.ShapeDtypeStruct((B, S, D), inputs.dtype),
    )(inputs, pos_table)
